# Initial kernel scaffold; baseline (speedup 1.0000x reference)
import functools

import jax
import jax.numpy as jnp
from jax import lax
from jax.experimental import pallas as pl
from jax.experimental.pallas import tpu as pltpu
from jax.experimental.pallas import tpu_sc as plsc

N = 10000
E = 320000
D = 128

NC = 2
NS = 16
NW = NC * NS
CH = 80
ROWS_E = E // CH
MROWS = ROWS_E // NW
DROWS = ROWS_E // NS
NZT = 10
ZR = N // NZT


def _rsqrt_newton(d):
    y = plsc.bitcast(jnp.int32(0x5F3759DF) - (plsc.bitcast(d, jnp.int32) >> 1),
                     jnp.float32)
    for _ in range(3):
        y = y * (1.5 - 0.5 * d * y * y)
    return jnp.where(d > 0.0, y, 0.0)


def _sc_body(src_hbm, dst_hbm, w_hbm, x_hbm, zn_hbm, znd_hbm, part_hbm,
             deg_sh, acc_sh, dsrc_v, dw_v, msrc_v, mdst_v, mw_v,
             dis_v, nw_v, rows_v, sem):
    c = lax.axis_index("c")
    s = lax.axis_index("s")
    wid = s * NC + c

    @pl.when(s < NZT)
    def _zero():
        pltpu.sync_copy(zn_hbm.at[pl.ds(s * ZR, ZR)],
                        deg_sh.at[pl.ds(s * ZR, ZR)])
        pltpu.sync_copy(znd_hbm.at[pl.ds(s * ZR, ZR)],
                        acc_sh.at[pl.ds(s * ZR, ZR)])

    pltpu.sync_copy(src_hbm.at[pl.ds(s * DROWS, DROWS)], dsrc_v)
    pltpu.sync_copy(w_hbm.at[pl.ds(s * DROWS, DROWS)], dw_v)
    plsc.subcore_barrier()

    def deg_body(j, _):
        pltpu.sync_copy(dw_v.at[j], deg_sh.at[dsrc_v.at[j]], add=True)
        return 0
    lax.fori_loop(0, DROWS, deg_body, 0)
    plsc.subcore_barrier()

    pltpu.sync_copy(deg_sh, dis_v)
    def rs_body(i, _):
        sl = pl.ds(i * 16, 16)
        dis_v[sl] = _rsqrt_newton(dis_v[sl])
        return 0
    lax.fori_loop(0, N // 16, rs_body, 0)

    pltpu.sync_copy(src_hbm.at[pl.ds(wid * MROWS, MROWS)], msrc_v)
    pltpu.sync_copy(dst_hbm.at[pl.ds(wid * MROWS, MROWS)], mdst_v)
    pltpu.sync_copy(w_hbm.at[pl.ds(wid * MROWS, MROWS)], mw_v)

    def msg_body(j, _):
        for k in range(CH // 16):
            sl = pl.ds(k * 16, 16)
            sidx = msrc_v[j, sl]
            didx = mdst_v[j, sl]
            wv = mw_v[j, sl]
            nw_v[sl] = -(plsc.load_gather(dis_v, [sidx]) * wv
                         * plsc.load_gather(dis_v, [didx]))
        pltpu.async_copy(x_hbm.at[msrc_v.at[j]], rows_v, sem).wait()
        def row_body(r, _):
            sc = nw_v[r]
            for k in range(D // 16):
                sl = pl.ds(k * 16, 16)
                rows_v[r, sl] = rows_v[r, sl] * sc
            return 0
        lax.fori_loop(0, CH, row_body, 0)
        pltpu.sync_copy(rows_v, acc_sh.at[mdst_v.at[j]], add=True)
        return 0
    lax.fori_loop(0, MROWS, msg_body, 0)
    plsc.subcore_barrier()

    @pl.when(s < NZT)
    def _out():
        pltpu.sync_copy(acc_sh.at[pl.ds(s * ZR, ZR)],
                        part_hbm.at[c, pl.ds(s * ZR, ZR)])


_sc_kernel = functools.partial(
    pl.kernel,
    out_type=jax.ShapeDtypeStruct((NC, N, D), jnp.float32),
    mesh=plsc.VectorSubcoreMesh(core_axis_name="c", subcore_axis_name="s"),
    scratch_types=[
        pltpu.VMEM_SHARED((N,), jnp.float32),
        pltpu.VMEM_SHARED((N, D), jnp.float32),
        pltpu.VMEM((DROWS, CH), jnp.int32),
        pltpu.VMEM((DROWS, CH), jnp.float32),
        pltpu.VMEM((MROWS, CH), jnp.int32),
        pltpu.VMEM((MROWS, CH), jnp.int32),
        pltpu.VMEM((MROWS, CH), jnp.float32),
        pltpu.VMEM((N,), jnp.float32),
        pltpu.VMEM((CH,), jnp.float32),
        pltpu.VMEM((CH, D), jnp.float32),
        pltpu.SemaphoreType.DMA,
    ],
)(_sc_body)


def _tc_body(x_ref, part_ref, wx_ref, bg_ref, wco_ref, wlin_ref, blin_ref,
             out_ref):
    tx1 = part_ref[0] + part_ref[1]
    xb = x_ref[...]
    g = (jnp.dot(xb, wx_ref[:D, :], preferred_element_type=jnp.float32)
         + jnp.dot(tx1, wx_ref[D:, :], preferred_element_type=jnp.float32)
         + bg_ref[...])
    gi = g[:, :D]
    gc = g[:, D:2 * D]
    go = g[:, 2 * D:]
    gate_i = jax.nn.sigmoid(gi)
    gate_t = jnp.tanh(gc)
    c2 = gate_i * gate_t
    gate_o = jax.nn.sigmoid(go + wco_ref[...] * c2)
    h = jnp.maximum(gate_o * jnp.tanh(c2), 0.0)
    out_ref[...] = (jnp.sum(h * wlin_ref[...], axis=1, keepdims=True)
                    + blin_ref[...])


def _tc_call(x, part, wx, bg, wco, wlin_t, blin):
    blk = 2000
    grid = N // blk
    return pl.pallas_call(
        _tc_body,
        grid=(grid,),
        in_specs=[
            pl.BlockSpec((blk, D), lambda i: (i, 0)),
            pl.BlockSpec((NC, blk, D), lambda i: (0, i, 0)),
            pl.BlockSpec((2 * D, 3 * D), lambda i: (0, 0)),
            pl.BlockSpec((1, 3 * D), lambda i: (0, 0)),
            pl.BlockSpec((1, D), lambda i: (0, 0)),
            pl.BlockSpec((1, D), lambda i: (0, 0)),
            pl.BlockSpec((1, 1), lambda i: (0, 0)),
        ],
        out_specs=pl.BlockSpec((blk, 1), lambda i: (i, 0)),
        out_shape=jax.ShapeDtypeStruct((N, 1), jnp.float32),
    )(x, part, wx, bg, wco, wlin_t, blin)


def kernel(x, edge_index, edge_weight, params):
    src = edge_index[0].reshape(ROWS_E, CH)
    dst = edge_index[1].reshape(ROWS_E, CH)
    w = edge_weight.reshape(ROWS_E, CH)
    zn = jnp.zeros((N,), jnp.float32)
    znd = jnp.zeros((N, D), jnp.float32)

    part = _sc_kernel(src, dst, w, x, zn, znd)

    wx = jnp.concatenate([
        jnp.concatenate([params["Wx0_i"], params["Wx0_c"], params["Wx0_o"]],
                        axis=1),
        jnp.concatenate([params["Wx1_i"], params["Wx1_c"], params["Wx1_o"]],
                        axis=1),
    ], axis=0)
    bg = jnp.concatenate([
        (params["bx_" + g] + params["bh_" + g])[None, :] + params["b_" + g]
        for g in ("i", "c", "o")
    ], axis=1)
    wlin_t = params["W_lin"].reshape(1, D)
    blin = params["b_lin"].reshape(1, 1)
    return _tc_call(x, part, wx, bg, params["wc_o"], wlin_t, blin)


# speedup vs baseline: 21.0877x; 21.0877x over previous
import functools

import jax
import jax.numpy as jnp
from jax import lax
from jax.experimental import pallas as pl
from jax.experimental.pallas import tpu as pltpu
from jax.experimental.pallas import tpu_sc as plsc

N = 10000
E = 320000
D = 128

NC = 2
NS = 16
NW = NC * NS
CH = 80
ROWS_E = E // CH
MROWS = ROWS_E // NW
RB = 25
NB = MROWS // RB
NZT = 10
ZR = N // NZT
SR = 40


def _deg_body(src_hbm, w_hbm, zn_hbm, degp_hbm, deg_sh, src_v, w_v, z_v):
    c = lax.axis_index("c")
    s = lax.axis_index("s")
    wid = s * NC + c

    @pl.when(s < NZT)
    def _zero():
        pltpu.sync_copy(zn_hbm.at[pl.ds(0, ZR)], z_v)
        pltpu.sync_copy(z_v, deg_sh.at[pl.ds(s * ZR, ZR)])

    plsc.subcore_barrier()

    def deg_blk(b, _):
        pltpu.sync_copy(src_hbm.at[wid, b], src_v)
        pltpu.sync_copy(w_hbm.at[wid, b], w_v)
        def deg_chunk(j, _):
            pltpu.sync_copy(w_v.at[j], deg_sh.at[src_v.at[j]], add=True)
            return 0
        lax.fori_loop(0, RB, deg_chunk, 0)
        return 0
    lax.fori_loop(0, NB, deg_blk, 0)
    plsc.subcore_barrier()

    @pl.when(s < NZT)
    def _out():
        pltpu.sync_copy(deg_sh.at[pl.ds(s * ZR, ZR)], z_v)
        pltpu.sync_copy(z_v, degp_hbm.at[c, s, 0])


_deg_kernel = functools.partial(
    pl.kernel,
    out_type=jax.ShapeDtypeStruct((NC, NZT, 1, ZR), jnp.float32),
    mesh=plsc.VectorSubcoreMesh(core_axis_name="c", subcore_axis_name="s"),
    scratch_types=[
        pltpu.VMEM_SHARED((N,), jnp.float32),
        pltpu.VMEM((RB, CH), jnp.int32),
        pltpu.VMEM((RB, CH), jnp.float32),
        pltpu.VMEM((ZR,), jnp.float32),
    ],
)(_deg_body)


def _prep_body(degp_ref, x_ref, xt_ref, dis_ref):
    blk = x_ref.shape[0]
    deg = degp_ref[0, 0] + degp_ref[0, 1]
    dis = jnp.where(deg > 0.0, lax.rsqrt(deg), 0.0).reshape(blk, 1)
    xt_ref[...] = x_ref[...] * (-dis)
    dis_ref[...] = dis


def _prep_call(degp, x):
    blk = 2000
    degp3 = degp.reshape(NC, N // blk, blk).transpose(1, 0, 2)
    return pl.pallas_call(
        _prep_body,
        grid=(N // blk,),
        in_specs=[
            pl.BlockSpec((1, NC, blk), lambda i: (i, 0, 0)),
            pl.BlockSpec((blk, D), lambda i: (i, 0)),
        ],
        out_specs=[
            pl.BlockSpec((blk, D), lambda i: (i, 0)),
            pl.BlockSpec((blk, 1), lambda i: (i, 0)),
        ],
        out_shape=[
            jax.ShapeDtypeStruct((N, D), jnp.float32),
            jax.ShapeDtypeStruct((N, 1), jnp.float32),
        ],
    )(degp3, x)


def _msg_body(src_hbm, dst_hbm, w_hbm, xt_hbm, znd_hbm, part_hbm,
              acc_sh, src_v, dst_v, w_v, rows_v, sem):
    c = lax.axis_index("c")
    s = lax.axis_index("s")
    wid = s * NC + c

    @pl.when(s < NZT)
    def _zero():
        pltpu.sync_copy(znd_hbm.at[pl.ds(0, SR)], rows_v.at[pl.ds(0, SR)])
        for t in range(ZR // SR):
            pltpu.sync_copy(rows_v.at[pl.ds(0, SR)],
                            acc_sh.at[pl.ds(s * ZR + t * SR, SR)])

    plsc.subcore_barrier()

    def msg_blk(b, _):
        pltpu.sync_copy(src_hbm.at[wid, b], src_v)
        pltpu.sync_copy(dst_hbm.at[wid, b], dst_v)
        pltpu.sync_copy(w_hbm.at[wid, b], w_v)
        def msg_chunk(j, _):
            pltpu.async_copy(xt_hbm.at[src_v.at[j]], rows_v, sem).wait()
            def row_blk(m, _):
                w16 = w_v[j, pl.ds(m * 16, 16)]
                for l in range(16):
                    r = m * 16 + l
                    sc = w16[l]
                    for k in range(D // 16):
                        sl = pl.ds(k * 16, 16)
                        rows_v[r, sl] = rows_v[r, sl] * sc
                return 0
            lax.fori_loop(0, CH // 16, row_blk, 0)
            pltpu.sync_copy(rows_v, acc_sh.at[dst_v.at[j]], add=True)
            return 0
        lax.fori_loop(0, RB, msg_chunk, 0)
        return 0
    lax.fori_loop(0, NB, msg_blk, 0)
    plsc.subcore_barrier()

    @pl.when(s < NZT)
    def _out():
        for t in range(ZR // SR):
            pltpu.sync_copy(acc_sh.at[pl.ds(s * ZR + t * SR, SR)],
                            rows_v.at[pl.ds(0, SR)])
            pltpu.sync_copy(rows_v.at[pl.ds(0, SR)],
                            part_hbm.at[c, pl.ds(s * ZR + t * SR, SR)])


_msg_kernel = functools.partial(
    pl.kernel,
    out_type=jax.ShapeDtypeStruct((NC, N, D), jnp.float32),
    mesh=plsc.VectorSubcoreMesh(core_axis_name="c", subcore_axis_name="s"),
    scratch_types=[
        pltpu.VMEM_SHARED((N, D), jnp.float32),
        pltpu.VMEM((RB, CH), jnp.int32),
        pltpu.VMEM((RB, CH), jnp.int32),
        pltpu.VMEM((RB, CH), jnp.float32),
        pltpu.VMEM((CH, D), jnp.float32),
        pltpu.SemaphoreType.DMA,
    ],
)(_msg_body)


def _final_body(x_ref, part_ref, dis_ref, wx_ref, bg_ref, wco_ref, wlin_ref,
                blin_ref, out_ref):
    tx1 = (part_ref[0] + part_ref[1]) * dis_ref[...]
    g = (jnp.dot(x_ref[...], wx_ref[:D, :], preferred_element_type=jnp.float32)
         + jnp.dot(tx1, wx_ref[D:, :], preferred_element_type=jnp.float32)
         + bg_ref[...])
    gate_i = jax.nn.sigmoid(g[:, :D])
    gate_t = jnp.tanh(g[:, D:2 * D])
    c2 = gate_i * gate_t
    gate_o = jax.nn.sigmoid(g[:, 2 * D:] + wco_ref[...] * c2)
    h = jnp.maximum(gate_o * jnp.tanh(c2), 0.0)
    out_ref[...] = (jnp.sum(h * wlin_ref[...], axis=1, keepdims=True)
                    + blin_ref[...])


def _final_call(x, part, dis, wx, bg, wco, wlin_t, blin):
    blk = 2000
    return pl.pallas_call(
        _final_body,
        grid=(N // blk,),
        in_specs=[
            pl.BlockSpec((blk, D), lambda i: (i, 0)),
            pl.BlockSpec((NC, blk, D), lambda i: (0, i, 0)),
            pl.BlockSpec((blk, 1), lambda i: (i, 0)),
            pl.BlockSpec((2 * D, 3 * D), lambda i: (0, 0)),
            pl.BlockSpec((1, 3 * D), lambda i: (0, 0)),
            pl.BlockSpec((1, D), lambda i: (0, 0)),
            pl.BlockSpec((1, D), lambda i: (0, 0)),
            pl.BlockSpec((1, 1), lambda i: (0, 0)),
        ],
        out_specs=pl.BlockSpec((blk, 1), lambda i: (i, 0)),
        out_shape=jax.ShapeDtypeStruct((N, 1), jnp.float32),
    )(x, part, dis, wx, bg, wco, wlin_t, blin)


def kernel(x, edge_index, edge_weight, params):
    src_m = edge_index[0].reshape(NW, NB, RB, CH)
    dst_m = edge_index[1].reshape(NW, NB, RB, CH)
    w_m = edge_weight.reshape(NW, NB, RB, CH)
    zn = jnp.zeros((N,), jnp.float32)
    znd = jnp.zeros((N, D), jnp.float32)

    degp = _deg_kernel(src_m, w_m, zn).reshape(NC, N)
    xt, dis = _prep_call(degp, x)
    part = _msg_kernel(src_m, dst_m, w_m, xt, znd)

    wx = jnp.concatenate([
        jnp.concatenate([params["Wx0_i"], params["Wx0_c"], params["Wx0_o"]],
                        axis=1),
        jnp.concatenate([params["Wx1_i"], params["Wx1_c"], params["Wx1_o"]],
                        axis=1),
    ], axis=0)
    bg = jnp.concatenate([
        (params["bx_" + g] + params["bh_" + g])[None, :] + params["b_" + g]
        for g in ("i", "c", "o")
    ], axis=1)
    wlin_t = params["W_lin"].reshape(1, D)
    blin = params["b_lin"].reshape(1, 1)
    return _final_call(x, part, dis, wx, bg, params["wc_o"], wlin_t, blin)


# speedup vs baseline: 28.6669x; 1.3594x over previous
import functools

import jax
import jax.numpy as jnp
from jax import lax
from jax.experimental import pallas as pl
from jax.experimental.pallas import tpu as pltpu
from jax.experimental.pallas import tpu_sc as plsc

N = 10000
E = 320000
D = 128

NC = 2
NS = 16
NW = NC * NS
CH = 80
ROWS_E = E // CH
MROWS = ROWS_E // NW
RB = 25
NB = MROWS // RB
NZT = 10
ZR = N // NZT
SR = 40


def _deg_body(src_hbm, w_hbm, zn_hbm, degp_hbm, deg_sh, src_v, w_v, z_v):
    c = lax.axis_index("c")
    s = lax.axis_index("s")
    wid = s * NC + c

    @pl.when(s < NZT)
    def _zero():
        pltpu.sync_copy(zn_hbm.at[pl.ds(0, ZR)], z_v)
        pltpu.sync_copy(z_v, deg_sh.at[pl.ds(s * ZR, ZR)])

    plsc.subcore_barrier()

    def deg_blk(b, _):
        pltpu.sync_copy(src_hbm.at[wid, b], src_v)
        pltpu.sync_copy(w_hbm.at[wid, b], w_v)
        def deg_chunk(j, _):
            pltpu.sync_copy(w_v.at[j], deg_sh.at[src_v.at[j]], add=True)
            return 0
        lax.fori_loop(0, RB, deg_chunk, 0)
        return 0
    lax.fori_loop(0, NB, deg_blk, 0)
    plsc.subcore_barrier()

    @pl.when(s < NZT)
    def _out():
        pltpu.sync_copy(deg_sh.at[pl.ds(s * ZR, ZR)], z_v)
        pltpu.sync_copy(z_v, degp_hbm.at[c, s, 0])


_deg_kernel = functools.partial(
    pl.kernel,
    out_type=jax.ShapeDtypeStruct((NC, NZT, 1, ZR), jnp.float32),
    mesh=plsc.VectorSubcoreMesh(core_axis_name="c", subcore_axis_name="s"),
    scratch_types=[
        pltpu.VMEM_SHARED((N,), jnp.float32),
        pltpu.VMEM((RB, CH), jnp.int32),
        pltpu.VMEM((RB, CH), jnp.float32),
        pltpu.VMEM((ZR,), jnp.float32),
    ],
)(_deg_body)


def _prep_body(degp_ref, x_ref, xt_ref, dis_ref):
    blk = x_ref.shape[0]
    deg = degp_ref[0, 0] + degp_ref[0, 1]
    dis = jnp.where(deg > 0.0, lax.rsqrt(deg), 0.0).reshape(blk, 1)
    xt_ref[...] = x_ref[...] * (-dis)
    dis_ref[...] = dis


def _prep_call(degp, x):
    blk = 2000
    degp3 = degp.reshape(NC, N // blk, blk).transpose(1, 0, 2)
    return pl.pallas_call(
        _prep_body,
        grid=(N // blk,),
        in_specs=[
            pl.BlockSpec((1, NC, blk), lambda i: (i, 0, 0)),
            pl.BlockSpec((blk, D), lambda i: (i, 0)),
        ],
        out_specs=[
            pl.BlockSpec((blk, D), lambda i: (i, 0)),
            pl.BlockSpec((blk, 1), lambda i: (i, 0)),
        ],
        out_shape=[
            jax.ShapeDtypeStruct((N, D), jnp.float32),
            jax.ShapeDtypeStruct((N, 1), jnp.float32),
        ],
    )(degp3, x)


def _msg_body(src_hbm, dst_hbm, w_hbm, xt_hbm, znd_hbm, part_hbm,
              acc_sh, src_v, dst_v, w_v, b0, b1, b2,
              gs0, gs1, gs2, ss0, ss1, ss2):
    c = lax.axis_index("c")
    s = lax.axis_index("s")
    wid = s * NC + c

    @pl.when(s < NZT)
    def _zero():
        pltpu.sync_copy(znd_hbm.at[pl.ds(0, SR)], b0.at[pl.ds(0, SR)])
        for t in range(ZR // SR):
            pltpu.sync_copy(b0.at[pl.ds(0, SR)],
                            acc_sh.at[pl.ds(s * ZR + t * SR, SR)])

    plsc.subcore_barrier()

    def _scale(j, buf):
        def row_blk(m, _):
            w16 = w_v[j, pl.ds(m * 16, 16)]
            for l in range(16):
                r = m * 16 + l
                sc = w16[l]
                for k in range(D // 16):
                    sl = pl.ds(k * 16, 16)
                    buf[r, sl] = buf[r, sl] * sc
            return 0
        lax.fori_loop(0, CH // 16, row_blk, 0)

    def _gather(t, buf, gsem):
        pltpu.async_copy(xt_hbm.at[src_v.at[t]], buf, gsem)

    def _wait_gather(t, buf, gsem):
        pltpu.make_async_copy(xt_hbm.at[src_v.at[t]], buf, gsem).wait()

    def _scatter(t, buf, ssem):
        pltpu.async_copy(buf, acc_sh.at[dst_v.at[t]], ssem, add=True)

    def _wait_scatter(t, buf, ssem):
        pltpu.make_async_copy(buf, acc_sh.at[dst_v.at[t]], ssem).wait()

    bufs = (b0, b1, b2)
    gsems = (gs0, gs1, gs2)
    ssems = (ss0, ss1, ss2)

    def _chunk(t, q, last):
        qn = (q + 1) % 3
        _wait_gather(t, bufs[q], gsems[q])
        @pl.when(t >= 2)
        def _():
            _wait_scatter(t - 2, bufs[qn], ssems[qn])
        if not last:
            _gather(t + 1, bufs[qn], gsems[qn])
        _scale(t, bufs[q])
        _scatter(t, bufs[q], ssems[q])

    def msg_blk(b, _):
        pltpu.sync_copy(src_hbm.at[wid, b], src_v)
        pltpu.sync_copy(dst_hbm.at[wid, b], dst_v)
        pltpu.sync_copy(w_hbm.at[wid, b], w_v)
        _gather(0, b0, gs0)

        def triple(i, _):
            t0 = 3 * i
            for q in range(3):
                _chunk(t0 + q, q, False)
            return 0
        lax.fori_loop(0, (RB - 1) // 3, triple, 0)

        t_last = RB - 1
        _chunk(t_last, 0, True)
        _wait_scatter(t_last - 1, bufs[2], ssems[2])
        _wait_scatter(t_last, bufs[0], ssems[0])
        return 0
    lax.fori_loop(0, NB, msg_blk, 0)
    plsc.subcore_barrier()

    @pl.when(s < NZT)
    def _out():
        for t in range(ZR // SR):
            pltpu.sync_copy(acc_sh.at[pl.ds(s * ZR + t * SR, SR)],
                            b0.at[pl.ds(0, SR)])
            pltpu.sync_copy(b0.at[pl.ds(0, SR)],
                            part_hbm.at[c, pl.ds(s * ZR + t * SR, SR)])


_msg_kernel = functools.partial(
    pl.kernel,
    out_type=jax.ShapeDtypeStruct((NC, N, D), jnp.float32),
    mesh=plsc.VectorSubcoreMesh(core_axis_name="c", subcore_axis_name="s"),
    scratch_types=[
        pltpu.VMEM_SHARED((N, D), jnp.float32),
        pltpu.VMEM((RB, CH), jnp.int32),
        pltpu.VMEM((RB, CH), jnp.int32),
        pltpu.VMEM((RB, CH), jnp.float32),
        pltpu.VMEM((CH, D), jnp.float32),
        pltpu.VMEM((CH, D), jnp.float32),
        pltpu.VMEM((CH, D), jnp.float32),
        pltpu.SemaphoreType.DMA,
        pltpu.SemaphoreType.DMA,
        pltpu.SemaphoreType.DMA,
        pltpu.SemaphoreType.DMA,
        pltpu.SemaphoreType.DMA,
        pltpu.SemaphoreType.DMA,
    ],
)(_msg_body)


def _final_body(x_ref, part_ref, dis_ref, wx_ref, bg_ref, wco_ref, wlin_ref,
                blin_ref, out_ref):
    tx1 = (part_ref[0] + part_ref[1]) * dis_ref[...]
    g = (jnp.dot(x_ref[...], wx_ref[:D, :], preferred_element_type=jnp.float32)
         + jnp.dot(tx1, wx_ref[D:, :], preferred_element_type=jnp.float32)
         + bg_ref[...])
    gate_i = jax.nn.sigmoid(g[:, :D])
    gate_t = jnp.tanh(g[:, D:2 * D])
    c2 = gate_i * gate_t
    gate_o = jax.nn.sigmoid(g[:, 2 * D:] + wco_ref[...] * c2)
    h = jnp.maximum(gate_o * jnp.tanh(c2), 0.0)
    out_ref[...] = (jnp.sum(h * wlin_ref[...], axis=1, keepdims=True)
                    + blin_ref[...])


def _final_call(x, part, dis, wx, bg, wco, wlin_t, blin):
    blk = 2000
    return pl.pallas_call(
        _final_body,
        grid=(N // blk,),
        in_specs=[
            pl.BlockSpec((blk, D), lambda i: (i, 0)),
            pl.BlockSpec((NC, blk, D), lambda i: (0, i, 0)),
            pl.BlockSpec((blk, 1), lambda i: (i, 0)),
            pl.BlockSpec((2 * D, 3 * D), lambda i: (0, 0)),
            pl.BlockSpec((1, 3 * D), lambda i: (0, 0)),
            pl.BlockSpec((1, D), lambda i: (0, 0)),
            pl.BlockSpec((1, D), lambda i: (0, 0)),
            pl.BlockSpec((1, 1), lambda i: (0, 0)),
        ],
        out_specs=pl.BlockSpec((blk, 1), lambda i: (i, 0)),
        out_shape=jax.ShapeDtypeStruct((N, 1), jnp.float32),
    )(x, part, dis, wx, bg, wco, wlin_t, blin)


def kernel(x, edge_index, edge_weight, params):
    src_m = edge_index[0].reshape(NW, NB, RB, CH)
    dst_m = edge_index[1].reshape(NW, NB, RB, CH)
    w_m = edge_weight.reshape(NW, NB, RB, CH)
    zn = jnp.zeros((N,), jnp.float32)
    znd = jnp.zeros((N, D), jnp.float32)

    degp = _deg_kernel(src_m, w_m, zn).reshape(NC, N)
    xt, dis = _prep_call(degp, x)
    part = _msg_kernel(src_m, dst_m, w_m, xt, znd)

    wx = jnp.concatenate([
        jnp.concatenate([params["Wx0_i"], params["Wx0_c"], params["Wx0_o"]],
                        axis=1),
        jnp.concatenate([params["Wx1_i"], params["Wx1_c"], params["Wx1_o"]],
                        axis=1),
    ], axis=0)
    bg = jnp.concatenate([
        (params["bx_" + g] + params["bh_" + g])[None, :] + params["b_" + g]
        for g in ("i", "c", "o")
    ], axis=1)
    wlin_t = params["W_lin"].reshape(1, D)
    blin = params["b_lin"].reshape(1, 1)
    return _final_call(x, part, dis, wx, bg, params["wc_o"], wlin_t, blin)
